# baseline (speedup 1.0000x reference)
import functools

import jax
import jax.numpy as jnp
from jax import lax
from jax.experimental import pallas as pl
from jax.experimental.pallas import tpu as pltpu
from jax.experimental.pallas import tpu_sc as plsc

N = 10000
D = 128
H = 64
NC = 2
NS = 16
L = 16
NW = NC * NS
CHUNK = 128
STRIPE = 632
AGG_ROWS = NS * STRIPE
BR = 2000


def _relu(v):
    return jnp.maximum(v, 0.0)


def _dot(a, b):
    return jnp.dot(a, b, preferred_element_type=jnp.float32)


def _pre_body(x_ref, w1, b1, w2, b2, w3, b3, wl, bl, h_ref, t_ref):
    h = _relu(_dot(x_ref[...], w1[...]) + b1[...])
    h = _relu(_dot(h, w2[...]) + b2[...])
    h = _relu(_dot(h, w3[...]) + b3[...])
    h_ref[...] = h
    t_ref[...] = _relu(_dot(h, wl[...]) + bl[...])


def _pre_call(x, W1, b1, W2, b2, W3, b3, Wl, bl):
    grid = (N // BR,)
    row_blk = lambda r, c: pl.BlockSpec((r, c), lambda i: (i, 0))
    full = lambda r, c: pl.BlockSpec((r, c), lambda i: (0, 0))
    return pl.pallas_call(
        _pre_body,
        grid=grid,
        in_specs=[
            row_blk(BR, D),
            full(D, H), full(1, H),
            full(H, H), full(1, H),
            full(H, H), full(1, H),
            full(H, H), full(1, H),
        ],
        out_specs=[
            pl.BlockSpec((BR, H), lambda i: (i, 0)),
            pl.BlockSpec((BR, H), lambda i: (i, 0)),
        ],
        out_shape=[
            jax.ShapeDtypeStruct((N, H), jnp.float32),
            jax.ShapeDtypeStruct((N, H), jnp.float32),
        ],
    )(x, W1, b1.reshape(1, H), W2, b2.reshape(1, H), W3, b3.reshape(1, H),
      Wl, bl.reshape(1, H))


def _sc_agg_body(t_hbm, src_hbm, dst_hbm, zeros_hbm, out_hbm,
                 src_v, dst_v, rows_v, zbuf_v, agg_sh, sem0, sem1):
    kch = src_v.shape[0]
    cid = lax.axis_index("c")
    sid = lax.axis_index("s")
    wid = sid * NC + cid

    pltpu.sync_copy(zeros_hbm, zbuf_v)
    base = pl.multiple_of(sid * STRIPE, 8)
    nfull = STRIPE // CHUNK
    for k in range(nfull):
        pltpu.sync_copy(zbuf_v, agg_sh.at[pl.ds(base + k * CHUNK, CHUNK)])
    rem = STRIPE - nfull * CHUNK
    if rem:
        pltpu.sync_copy(zbuf_v.at[pl.ds(0, rem)],
                        agg_sh.at[pl.ds(base + nfull * CHUNK, rem)])
    plsc.subcore_barrier()

    pltpu.sync_copy(src_hbm.at[wid], src_v)
    pltpu.sync_copy(dst_hbm.at[wid], dst_v)

    def gather(c, b, s):
        return pltpu.async_copy(t_hbm.at[src_v.at[c]], rows_v.at[b], s)

    def gather_wait(c, b, s):
        pltpu.make_async_copy(t_hbm.at[src_v.at[c]], rows_v.at[b], s).wait()

    def scatter(c, b):
        pltpu.sync_copy(rows_v.at[b], agg_sh.at[dst_v.at[c]], add=True)

    gather(0, 0, sem0)

    def pair_body(jj, carry):
        c0 = 2 * jj
        c1 = c0 + 1
        gather_wait(c0, 0, sem0)
        gather(c1, 1, sem1)
        scatter(c0, 0)
        gather_wait(c1, 1, sem1)
        nxt = jnp.minimum(c0 + 2, kch - 2)
        gather(nxt, 0, sem0)
        scatter(c1, 1)
        return carry

    lax.fori_loop(0, kch // 2, pair_body, 0)
    gather_wait(kch - 2, 0, sem0)
    plsc.subcore_barrier()

    out_base = pl.multiple_of(jnp.minimum(base, N - STRIPE), 8)
    pltpu.sync_copy(agg_sh.at[pl.ds(out_base, STRIPE)],
                    out_hbm.at[cid, pl.ds(out_base, STRIPE)])


def _sc_agg_call(t, srcb, dstb, zeros_blk):
    kch = srcb.shape[1]
    mesh = plsc.VectorSubcoreMesh(core_axis_name="c", subcore_axis_name="s",
                                  num_cores=NC, num_subcores=NS)
    fn = functools.partial(
        pl.kernel,
        out_type=jax.ShapeDtypeStruct((NC, N, H), jnp.float32),
        mesh=mesh,
        scratch_types=[
            pltpu.VMEM((kch, CHUNK), jnp.int32),
            pltpu.VMEM((kch, CHUNK), jnp.int32),
            pltpu.VMEM((2, CHUNK, H), jnp.float32),
            pltpu.VMEM((CHUNK, H), jnp.float32),
            pltpu.VMEM_SHARED((AGG_ROWS, H), jnp.float32),
            pltpu.SemaphoreType.DMA,
            pltpu.SemaphoreType.DMA,
        ],
        compiler_params=pltpu.CompilerParams(use_tc_tiling_on_sc=False),
    )(_sc_agg_body)
    return fn(t, srcb, dstb, zeros_blk)


def _post_body(h_ref, agg_ref, p1, pb1, p2, pb2, p3, pb3, o_ref):
    a = agg_ref[0] + agg_ref[1]
    cat = jnp.concatenate([h_ref[...], a], axis=1)
    u = _relu(_dot(cat, p1[...]) + pb1[...])
    u = _relu(_dot(u, p2[...]) + pb2[...])
    o_ref[...] = _relu(_dot(u, p3[...]) + pb3[...])


def _post_call(h_pre, agg2, P1, pb1, P2, pb2, P3, pb3):
    grid = (N // BR,)
    full = lambda r, c: pl.BlockSpec((r, c), lambda i: (0, 0))
    return pl.pallas_call(
        _post_body,
        grid=grid,
        in_specs=[
            pl.BlockSpec((BR, H), lambda i: (i, 0)),
            pl.BlockSpec((NC, BR, H), lambda i: (0, i, 0)),
            full(2 * H, H), full(1, H),
            full(H, H), full(1, H),
            full(H, D), full(1, D),
        ],
        out_specs=pl.BlockSpec((BR, D), lambda i: (i, 0)),
        out_shape=jax.ShapeDtypeStruct((N, D), jnp.float32),
    )(h_pre, agg2, P1, pb1.reshape(1, H), P2, pb2.reshape(1, H),
      P3, pb3.reshape(1, D))


def kernel(x, edge_index, W1, b1, W2, b2, W3, b3, Wl, bl,
           P1, pb1, P2, pb2, P3, pb3):
    E = edge_index.shape[1]
    kch = -(-E // (NW * CHUNK))
    kch += kch % 2
    pad_e = NW * CHUNK * kch - E
    src = edge_index[0].astype(jnp.int32)
    dst = edge_index[1].astype(jnp.int32)
    src = jnp.concatenate([src, jnp.zeros((pad_e,), jnp.int32)])
    dst = jnp.concatenate([dst, jnp.full((pad_e,), N, jnp.int32)])
    srcb = src.reshape(NW, kch, CHUNK)
    dstb = dst.reshape(NW, kch, CHUNK)
    zeros_blk = jnp.zeros((CHUNK, H), jnp.float32)

    h_pre, t = _pre_call(x, W1, b1, W2, b2, W3, b3, Wl, bl)
    agg2 = _sc_agg_call(t, srcb, dstb, zeros_blk)
    return _post_call(h_pre, agg2, P1, pb1, P2, pb2, P3, pb3)


# speedup vs baseline: 1.6407x; 1.6407x over previous
import functools

import jax
import jax.numpy as jnp
from jax import lax
from jax.experimental import pallas as pl
from jax.experimental.pallas import tpu as pltpu
from jax.experimental.pallas import tpu_sc as plsc

N = 10000
D = 128
H = 64
NC = 2
NS = 16
L = 16
NW = NC * NS
CHUNK = 128
STRIPE = 632
AGG_ROWS = NS * STRIPE
BR = 2000


def _relu(v):
    return jnp.maximum(v, 0.0)


def _dot(a, b):
    return jnp.dot(a, b, preferred_element_type=jnp.float32)


def _pre_body(x_ref, w1, b1, w2, b2, w3, b3, wl, bl, h_ref, t_ref):
    h = _relu(_dot(x_ref[...], w1[...]) + b1[...])
    h = _relu(_dot(h, w2[...]) + b2[...])
    h = _relu(_dot(h, w3[...]) + b3[...])
    h_ref[...] = h
    t_ref[...] = _relu(_dot(h, wl[...]) + bl[...])


def _pre_call(x, W1, b1, W2, b2, W3, b3, Wl, bl):
    grid = (N // BR,)
    row_blk = lambda r, c: pl.BlockSpec((r, c), lambda i: (i, 0))
    full = lambda r, c: pl.BlockSpec((r, c), lambda i: (0, 0))
    return pl.pallas_call(
        _pre_body,
        grid=grid,
        in_specs=[
            row_blk(BR, D),
            full(D, H), full(1, H),
            full(H, H), full(1, H),
            full(H, H), full(1, H),
            full(H, H), full(1, H),
        ],
        out_specs=[
            pl.BlockSpec((BR, H), lambda i: (i, 0)),
            pl.BlockSpec((BR, H), lambda i: (i, 0)),
        ],
        out_shape=[
            jax.ShapeDtypeStruct((N, H), jnp.float32),
            jax.ShapeDtypeStruct((N, H), jnp.float32),
        ],
    )(x, W1, b1.reshape(1, H), W2, b2.reshape(1, H), W3, b3.reshape(1, H),
      Wl, bl.reshape(1, H))


def _sc_agg_body(t_hbm, src_hbm, dst_hbm, zeros_hbm, out_hbm,
                 src_v, dst_v, rows_v, zbuf_v, t_sh, agg_sh):
    kch = src_v.shape[0]
    cid = lax.axis_index("c")
    sid = lax.axis_index("s")
    wid = sid * NC + cid

    pltpu.sync_copy(zeros_hbm, zbuf_v)
    base = pl.multiple_of(sid * STRIPE, 8)
    nfull = STRIPE // CHUNK
    for k in range(nfull):
        pltpu.sync_copy(zbuf_v, agg_sh.at[pl.ds(base + k * CHUNK, CHUNK)])
    rem = STRIPE - nfull * CHUNK
    if rem:
        pltpu.sync_copy(zbuf_v.at[pl.ds(0, rem)],
                        agg_sh.at[pl.ds(base + nfull * CHUNK, rem)])

    t_base = pl.multiple_of(jnp.minimum(base, N - STRIPE), 8)
    pltpu.sync_copy(t_hbm.at[pl.ds(t_base, STRIPE)],
                    t_sh.at[pl.ds(t_base, STRIPE)])

    pltpu.sync_copy(src_hbm.at[wid], src_v)
    pltpu.sync_copy(dst_hbm.at[wid], dst_v)
    plsc.subcore_barrier()

    def chunk_body(c, carry):
        pltpu.sync_copy(t_sh.at[src_v.at[c]], rows_v)
        pltpu.sync_copy(rows_v, agg_sh.at[dst_v.at[c]], add=True)
        return carry

    lax.fori_loop(0, kch, chunk_body, 0)
    plsc.subcore_barrier()

    out_base = pl.multiple_of(jnp.minimum(base, N - STRIPE), 8)
    pltpu.sync_copy(agg_sh.at[pl.ds(out_base, STRIPE)],
                    out_hbm.at[cid, pl.ds(out_base, STRIPE)])


def _sc_agg_call(t, srcb, dstb, zeros_blk):
    kch = srcb.shape[1]
    mesh = plsc.VectorSubcoreMesh(core_axis_name="c", subcore_axis_name="s",
                                  num_cores=NC, num_subcores=NS)
    fn = functools.partial(
        pl.kernel,
        out_type=jax.ShapeDtypeStruct((NC, N, H), jnp.float32),
        mesh=mesh,
        scratch_types=[
            pltpu.VMEM((kch, CHUNK), jnp.int32),
            pltpu.VMEM((kch, CHUNK), jnp.int32),
            pltpu.VMEM((CHUNK, H), jnp.float32),
            pltpu.VMEM((CHUNK, H), jnp.float32),
            pltpu.VMEM_SHARED((AGG_ROWS, H), jnp.float32),
            pltpu.VMEM_SHARED((AGG_ROWS, H), jnp.float32),
        ],
        compiler_params=pltpu.CompilerParams(use_tc_tiling_on_sc=False),
    )(_sc_agg_body)
    return fn(t, srcb, dstb, zeros_blk)


def _post_body(h_ref, agg_ref, p1, pb1, p2, pb2, p3, pb3, o_ref):
    a = agg_ref[0] + agg_ref[1]
    cat = jnp.concatenate([h_ref[...], a], axis=1)
    u = _relu(_dot(cat, p1[...]) + pb1[...])
    u = _relu(_dot(u, p2[...]) + pb2[...])
    o_ref[...] = _relu(_dot(u, p3[...]) + pb3[...])


def _post_call(h_pre, agg2, P1, pb1, P2, pb2, P3, pb3):
    grid = (N // BR,)
    full = lambda r, c: pl.BlockSpec((r, c), lambda i: (0, 0))
    return pl.pallas_call(
        _post_body,
        grid=grid,
        in_specs=[
            pl.BlockSpec((BR, H), lambda i: (i, 0)),
            pl.BlockSpec((NC, BR, H), lambda i: (0, i, 0)),
            full(2 * H, H), full(1, H),
            full(H, H), full(1, H),
            full(H, D), full(1, D),
        ],
        out_specs=pl.BlockSpec((BR, D), lambda i: (i, 0)),
        out_shape=jax.ShapeDtypeStruct((N, D), jnp.float32),
    )(h_pre, agg2, P1, pb1.reshape(1, H), P2, pb2.reshape(1, H),
      P3, pb3.reshape(1, D))


def kernel(x, edge_index, W1, b1, W2, b2, W3, b3, Wl, bl,
           P1, pb1, P2, pb2, P3, pb3):
    E = edge_index.shape[1]
    kch = -(-E // (NW * CHUNK))
    pad_e = NW * CHUNK * kch - E
    src = edge_index[0].astype(jnp.int32)
    dst = edge_index[1].astype(jnp.int32)
    src = jnp.concatenate([src, jnp.zeros((pad_e,), jnp.int32)])
    dst = jnp.concatenate([dst, jnp.full((pad_e,), N, jnp.int32)])
    srcb = src.reshape(NW, kch, CHUNK)
    dstb = dst.reshape(NW, kch, CHUNK)
    zeros_blk = jnp.zeros((CHUNK, H), jnp.float32)

    h_pre, t = _pre_call(x, W1, b1, W2, b2, W3, b3, Wl, bl)
    agg2 = _sc_agg_call(t, srcb, dstb, zeros_blk)
    return _post_call(h_pre, agg2, P1, pb1, P2, pb2, P3, pb3)
